# baseline (speedup 1.0000x reference)
import jax
import jax.numpy as jnp
from jax import lax
from jax.experimental import pallas as pl
from jax.experimental.pallas import tpu as pltpu
from jax.experimental.pallas import tpu_sc as plsc

N = 4096
E = 65536
D = 128
H = 64
C = 7
B = 64
NA = 32
NCUT = E // 2
SEL = 3 * B
HALF = SEL // 2
NPAD = 4224
NEG_BIG = -1e30

EDGES_PER_TILE = E // 16
CHUNK = 128
CHUNKS_PER_TILE = EDGES_PER_TILE // CHUNK


def _prep_kernel(x_ref, encw_ref, encb_ref, predw_ref, predb_ref, sel_ref,
                 aidx_ref, w_ref, src_ref, idx_ref,
                 m_ref, embed_ref, onehot_ref, logp_ref, psrc_ref):
    f32 = jnp.float32
    xw = x_ref[...] + idx_ref[0, 0]
    embed = jnp.maximum(
        jnp.dot(xw, encw_ref[...], preferred_element_type=f32)
        + encb_ref[...], 0.0)
    embed_ref[...] = embed

    logits = (jnp.dot(embed, predw_ref[...], preferred_element_type=f32)
              + predb_ref[...])
    colmask = lax.broadcasted_iota(jnp.int32, (N, 128), 1) < C
    ml = jnp.where(colmask, logits, jnp.float32(NEG_BIG))
    mx = jnp.max(ml, axis=1, keepdims=True)
    z = ml - mx
    s = jnp.sum(jnp.where(colmask, jnp.exp(z), jnp.float32(0.0)), axis=1, keepdims=True)
    logp_ref[...] = z - jnp.log(s)

    iota_n = lax.broadcasted_iota(jnp.int32, (C, NA, N), 2)
    eq = (iota_n == aidx_ref[...][:, :, None]).astype(f32)
    p_mat = jnp.sum(eq, axis=1)
    anchor = jnp.dot(p_mat, embed, preferred_element_type=f32,
                     precision=lax.Precision.HIGHEST) * (1.0 / NA)

    d2_cols = []
    for c in range(C):
        diff = embed - anchor[c:c + 1, :]
        d2_cols.append(jnp.sum(diff * diff, axis=1, keepdims=True))
    d2_cols.append(jnp.full((N, 1), 3e38, f32))
    d2 = jnp.concatenate(d2_cols, axis=1)
    mind = jnp.min(d2, axis=1, keepdims=True)
    iota8 = lax.broadcasted_iota(jnp.int32, (N, 8), 1)
    cand = jnp.where(d2 == mind, iota8, 8)
    cls = jnp.min(cand, axis=1, keepdims=True)
    onehot_ref[...] = (iota8 == cls).astype(f32)

    iota_v = lax.broadcasted_iota(jnp.int32, (2, NPAD, HALF), 1)
    m_ref[...] = (iota_v == sel_ref[...][:, None, :]).astype(f32)

    w = w_ref[...]

    def bs_body(_, carry):
        lo, hi = carry
        mid = (lo + hi) // 2
        tv = lax.bitcast_convert_type(jnp.full((1, 1), mid, jnp.int32), f32)
        cnt = jnp.sum((w >= tv).astype(f32))
        pred = cnt >= float(NCUT)
        return (jnp.where(pred, mid, lo), jnp.where(pred, hi, mid))

    lo, _ = lax.fori_loop(0, 31, bs_body,
                          (jnp.int32(0), jnp.int32(0x3F800000)))
    thresh = lax.bitcast_convert_type(jnp.full((1, 1), lo, jnp.int32), f32)
    psrc_ref[...] = jnp.where(w >= thresh, src_ref[...], jnp.int32(N))


def _hop_kernel(m_hbm, psrc_hbm, pdst_hbm, out_hbm,
                h1_sh, h2_sh, src_v, dst_v, rows_v, bounce_v, sem):
    cid = lax.axis_index("c")
    sid = lax.axis_index("s")

    pltpu.sync_copy(m_hbm.at[cid].at[pl.ds(sid * 264, 264)],
                    h1_sh.at[pl.ds(sid * 264, 264)])
    pltpu.sync_copy(psrc_hbm.at[sid], src_v)
    pltpu.sync_copy(pdst_hbm.at[sid], dst_v)
    plsc.subcore_barrier()

    for j in range(CHUNKS_PER_TILE):
        pltpu.async_copy(m_hbm.at[cid].at[src_v.at[jnp.int32(j)]], rows_v, sem).wait()
        pltpu.sync_copy(rows_v, h1_sh.at[dst_v.at[jnp.int32(j)]], add=True)
    plsc.subcore_barrier()

    pltpu.sync_copy(h1_sh.at[pl.ds(sid * 256, 256)], bounce_v)
    pltpu.sync_copy(bounce_v, h2_sh.at[pl.ds(sid * 256, 256)])
    plsc.subcore_barrier()

    for j in range(CHUNKS_PER_TILE):
        pltpu.async_copy(h1_sh.at[src_v.at[jnp.int32(j)]], rows_v, sem).wait()
        pltpu.sync_copy(rows_v, h2_sh.at[dst_v.at[jnp.int32(j)]], add=True)
    plsc.subcore_barrier()

    pltpu.sync_copy(h2_sh.at[pl.ds(sid * 256, 256)],
                    out_hbm.at[cid].at[pl.ds(sid * 256, 256)])


def _reduce_kernel(h2_ref, embed_ref, onehot_ref,
                   d1w_ref, d1b_ref, d2w_ref, d2b_ref, out_ref):
    f32 = jnp.float32
    hb0 = (h2_ref[0] > 0.0).astype(f32)
    hb1 = (h2_ref[1] > 0.0).astype(f32)
    embed = embed_ref[...]
    onehot = onehot_ref[...]
    dn = (((0,), (0,)), ((), ()))
    c0 = lax.dot_general(hb0, onehot, dn, preferred_element_type=f32,
                         precision=lax.Precision.HIGHEST)
    c1 = lax.dot_general(hb1, onehot, dn, preferred_element_type=f32,
                         precision=lax.Precision.HIGHEST)
    counts = jnp.concatenate([c0, c1], axis=0)

    accp = jnp.zeros((B, 32), f32)
    accn = jnp.zeros((B, 32), f32)
    for c in range(C):
        wh = embed * onehot[:, c:c + 1]
        s0 = lax.dot_general(hb0, wh, dn, preferred_element_type=f32,
                             precision=lax.Precision.HIGHEST)
        s1 = lax.dot_general(hb1, wh, dn, preferred_element_type=f32,
                             precision=lax.Precision.HIGHEST)
        sums = jnp.concatenate([s0, s1], axis=0)
        means = sums / jnp.maximum(counts[:, c:c + 1], 1.0)
        e_r = means[0:B]
        p_r = means[B:2 * B]
        n_r = means[2 * B:3 * B]
        dp = (e_r - p_r) * (e_r - p_r)
        dnn = (e_r - n_r) * (e_r - n_r)
        w_c = d1w_ref[c]
        accp = accp + jnp.dot(dp, w_c, preferred_element_type=f32,
                              precision=lax.Precision.HIGHEST)
        accn = accn + jnp.dot(dnn, w_c, preferred_element_type=f32,
                              precision=lax.Precision.HIGHEST)

    hp = jnp.maximum(accp + d1b_ref[...], 0.0)
    hn = jnp.maximum(accn + d1b_ref[...], 0.0)
    pos_l = jnp.sum(hp * d2w_ref[...], axis=1, keepdims=True) + d2b_ref[0, 0]
    neg_l = jnp.sum(hn * d2w_ref[...], axis=1, keepdims=True) + d2b_ref[0, 0]
    pad = jnp.zeros((B, 126), f32)
    out_ref[...] = jnp.concatenate([pos_l, neg_l, pad], axis=1)


def _prep_call(x, encw, encb, predw_pad, predb_pad, sel2, aidx, w2, src2, idxs):
    f32 = jnp.float32
    i32 = jnp.int32
    vspec = pl.BlockSpec(memory_space=pltpu.MemorySpace.VMEM)
    return pl.pallas_call(
        _prep_kernel,
        out_shape=[
            jax.ShapeDtypeStruct((2, NPAD, HALF), f32),
            jax.ShapeDtypeStruct((N, H), f32),
            jax.ShapeDtypeStruct((N, 8), f32),
            jax.ShapeDtypeStruct((N, 128), f32),
            jax.ShapeDtypeStruct((512, 128), i32),
        ],
        in_specs=[vspec] * 9 + [pl.BlockSpec(memory_space=pltpu.MemorySpace.SMEM)],
        out_specs=[vspec] * 5,
    )(x, encw, encb, predw_pad, predb_pad, sel2, aidx, w2, src2, idxs)


def _hop_call(m, psrc3, pdst3):
    mesh = plsc.VectorSubcoreMesh(core_axis_name="c", subcore_axis_name="s")
    f32 = jnp.float32
    i32 = jnp.int32
    return pl.kernel(
        _hop_kernel,
        out_type=jax.ShapeDtypeStruct((2, N, HALF), f32),
        mesh=mesh,
        scratch_types=[
            pltpu.VMEM_SHARED((NPAD, HALF), f32),
            pltpu.VMEM_SHARED((N, HALF), f32),
            pltpu.VMEM((CHUNKS_PER_TILE, CHUNK), i32),
            pltpu.VMEM((CHUNKS_PER_TILE, CHUNK), i32),
            pltpu.VMEM((CHUNK, HALF), f32),
            pltpu.VMEM((256, HALF), f32),
            pltpu.SemaphoreType.DMA,
        ],
        compiler_params=pltpu.CompilerParams(use_tc_tiling_on_sc=False),
    )(m, psrc3, pdst3)


def _reduce_call(h2, embed, onehot, d1w3, d1b, d2wr, d2b):
    vspec = pl.BlockSpec(memory_space=pltpu.MemorySpace.VMEM)
    return pl.pallas_call(
        _reduce_kernel,
        out_shape=jax.ShapeDtypeStruct((B, 128), jnp.float32),
        in_specs=[vspec] * 7,
        out_specs=vspec,
    )(h2, embed, onehot, d1w3, d1b, d2wr, d2b)


def kernel(x, edge_index, ego, pos, neg, edge_weight, anchor_idx, idx,
           enc_W, enc_b, pred_W, pred_b, dec1_W, dec1_b, dec2_W, dec2_b):
    f32 = jnp.float32
    i32 = jnp.int32
    src2 = edge_index[0].astype(i32).reshape(512, 128)
    dst = edge_index[1].astype(i32)
    w2 = edge_weight.astype(f32).reshape(512, 128)
    sel2 = jnp.concatenate([ego, pos, neg]).astype(i32).reshape(2, HALF)
    aidx = anchor_idx.astype(i32)
    idxs = jnp.asarray(idx, f32).reshape(1, 1)
    predw_pad = jnp.pad(pred_W.astype(f32), ((0, 0), (0, 128 - C)))
    predb_pad = jnp.pad(pred_b.astype(f32), (0, 128 - C)).reshape(1, 128)
    encb = enc_b.astype(f32).reshape(1, H)

    m, embed, onehot, logp_pad, psrc2 = _prep_call(
        x.astype(f32), enc_W.astype(f32), encb, predw_pad, predb_pad, sel2,
        aidx, w2, src2, idxs)

    psrc3 = psrc2.reshape(16, CHUNKS_PER_TILE, CHUNK)
    pdst3 = dst.reshape(16, CHUNKS_PER_TILE, CHUNK)
    h2 = _hop_call(m, psrc3, pdst3)

    d1w3 = dec1_W.astype(f32).reshape(C, H, 32)
    d1b = dec1_b.astype(f32).reshape(1, 32)
    d2wr = dec2_W.astype(f32).reshape(1, 32)
    d2b = dec2_b.astype(f32).reshape(1, 1)
    out2 = _reduce_call(h2, embed, onehot, d1w3, d1b, d2wr, d2b)

    return (out2[:, 0:1], out2[:, 1:2], logp_pad[:, :C])


# speedup vs baseline: 1.0126x; 1.0126x over previous
import jax
import jax.numpy as jnp
from jax import lax
from jax.experimental import pallas as pl
from jax.experimental.pallas import tpu as pltpu
from jax.experimental.pallas import tpu_sc as plsc

N = 4096
E = 65536
D = 128
H = 64
C = 7
B = 64
NA = 32
NCUT = E // 2
SEL = 3 * B
HALF = SEL // 2
NPAD = 4224
NEG_BIG = -1e30

EDGES_PER_TILE = E // 16
CHUNK = 128
CHUNKS_PER_TILE = EDGES_PER_TILE // CHUNK


def _prep_kernel(x_ref, encw_ref, encb_ref, predw_ref, predb_ref, sel_ref,
                 aidx_ref, w_ref, src_ref, idx_ref,
                 m_ref, embed_ref, onehot_ref, logp_ref, psrc_ref):
    f32 = jnp.float32
    xw = x_ref[...] + idx_ref[0, 0]
    embed = jnp.maximum(
        jnp.dot(xw, encw_ref[...], preferred_element_type=f32)
        + encb_ref[...], 0.0)
    embed_ref[...] = embed

    logits = (jnp.dot(embed, predw_ref[...], preferred_element_type=f32)
              + predb_ref[...])
    colmask = lax.broadcasted_iota(jnp.int32, (N, 128), 1) < C
    ml = jnp.where(colmask, logits, jnp.float32(NEG_BIG))
    mx = jnp.max(ml, axis=1, keepdims=True)
    z = ml - mx
    s = jnp.sum(jnp.where(colmask, jnp.exp(z), jnp.float32(0.0)), axis=1, keepdims=True)
    logp_ref[...] = z - jnp.log(s)

    iota_n = lax.broadcasted_iota(jnp.int32, (C, NA, N), 2)
    eq = (iota_n == aidx_ref[...][:, :, None]).astype(f32)
    p_mat = jnp.sum(eq, axis=1)
    anchor = jnp.dot(p_mat, embed, preferred_element_type=f32,
                     precision=lax.Precision.HIGHEST) * (1.0 / NA)

    d2_cols = []
    for c in range(C):
        diff = embed - anchor[c:c + 1, :]
        d2_cols.append(jnp.sum(diff * diff, axis=1, keepdims=True))
    d2_cols.append(jnp.full((N, 1), 3e38, f32))
    d2 = jnp.concatenate(d2_cols, axis=1)
    mind = jnp.min(d2, axis=1, keepdims=True)
    iota8 = lax.broadcasted_iota(jnp.int32, (N, 8), 1)
    cand = jnp.where(d2 == mind, iota8, 8)
    cls = jnp.min(cand, axis=1, keepdims=True)
    onehot_ref[...] = (iota8 == cls).astype(f32)

    iota_v = lax.broadcasted_iota(jnp.int32, (2, NPAD, HALF), 1)
    m_ref[...] = (iota_v == sel_ref[...][:, None, :]).astype(f32)

    w = w_ref[...]

    def bs_body(_, carry):
        lo, hi = carry
        mid = (lo + hi) // 2
        tv = lax.bitcast_convert_type(jnp.full((1, 1), mid, jnp.int32), f32)
        cnt = jnp.sum((w >= tv).astype(f32))
        pred = cnt >= float(NCUT)
        return (jnp.where(pred, mid, lo), jnp.where(pred, hi, mid))

    lo, _ = lax.fori_loop(0, 31, bs_body,
                          (jnp.int32(0), jnp.int32(0x3F800000)))
    thresh = lax.bitcast_convert_type(jnp.full((1, 1), lo, jnp.int32), f32)
    psrc_ref[...] = jnp.where(w >= thresh, src_ref[...], jnp.int32(N))


def _hop_kernel(m_hbm, psrc_hbm, pdst_hbm, out_hbm,
                h1_sh, h2_sh, src_v, dst_v, rows_v, bounce_v,
                g0, g1, s0, s1):
    cid = lax.axis_index("c")
    sid = lax.axis_index("s")
    gsem = (g0, g1)
    ssem = (s0, s1)

    @pl.when(sid < 15)
    def _():
        pltpu.sync_copy(m_hbm.at[cid].at[pl.ds(sid * 256, 256)],
                        h1_sh.at[pl.ds(sid * 256, 256)])

    @pl.when(sid == 15)
    def _():
        pltpu.sync_copy(m_hbm.at[cid].at[pl.ds(3840, 264)],
                        h1_sh.at[pl.ds(3840, 264)])

    pltpu.sync_copy(psrc_hbm.at[sid], src_v)
    pltpu.sync_copy(pdst_hbm.at[sid], dst_v)
    plsc.subcore_barrier()

    def run_pass(src_sh, dst_sh):
        gd = [None, None]
        sd = [None, None]
        gd[0] = pltpu.async_copy(src_sh.at[src_v.at[jnp.int32(0)]],
                                 rows_v.at[jnp.int32(0)], gsem[0])
        for j in range(CHUNKS_PER_TILE):
            b = j % 2
            if j + 1 < CHUNKS_PER_TILE:
                nb = (j + 1) % 2
                if sd[nb] is not None:
                    sd[nb].wait()
                gd[nb] = pltpu.async_copy(
                    src_sh.at[src_v.at[jnp.int32(j + 1)]],
                    rows_v.at[jnp.int32(nb)], gsem[nb])
            gd[b].wait()
            sd[b] = pltpu.async_copy(rows_v.at[jnp.int32(b)],
                                     dst_sh.at[dst_v.at[jnp.int32(j)]],
                                     ssem[b], add=True)
        sd[0].wait()
        sd[1].wait()

    run_pass(m_hbm.at[cid], h1_sh)
    plsc.subcore_barrier()

    pltpu.sync_copy(h1_sh.at[pl.ds(sid * 256, 256)], bounce_v)
    pltpu.sync_copy(bounce_v, h2_sh.at[pl.ds(sid * 256, 256)])
    plsc.subcore_barrier()

    run_pass(h1_sh, h2_sh)
    plsc.subcore_barrier()

    pltpu.sync_copy(h2_sh.at[pl.ds(sid * 256, 256)],
                    out_hbm.at[cid].at[pl.ds(sid * 256, 256)])


def _reduce_kernel(h2_ref, embed_ref, onehot_ref,
                   d1w_ref, d1b_ref, d2w_ref, d2b_ref, out_ref):
    f32 = jnp.float32
    hb0 = (h2_ref[0] > 0.0).astype(f32)
    hb1 = (h2_ref[1] > 0.0).astype(f32)
    embed = embed_ref[...]
    onehot = onehot_ref[...]
    dn = (((0,), (0,)), ((), ()))
    c0 = lax.dot_general(hb0, onehot, dn, preferred_element_type=f32,
                         precision=lax.Precision.HIGHEST)
    c1 = lax.dot_general(hb1, onehot, dn, preferred_element_type=f32,
                         precision=lax.Precision.HIGHEST)
    counts = jnp.concatenate([c0, c1], axis=0)

    accp = jnp.zeros((B, 32), f32)
    accn = jnp.zeros((B, 32), f32)
    for c in range(C):
        wh = embed * onehot[:, c:c + 1]
        s0 = lax.dot_general(hb0, wh, dn, preferred_element_type=f32,
                             precision=lax.Precision.HIGHEST)
        s1 = lax.dot_general(hb1, wh, dn, preferred_element_type=f32,
                             precision=lax.Precision.HIGHEST)
        sums = jnp.concatenate([s0, s1], axis=0)
        means = sums / jnp.maximum(counts[:, c:c + 1], 1.0)
        e_r = means[0:B]
        p_r = means[B:2 * B]
        n_r = means[2 * B:3 * B]
        dp = (e_r - p_r) * (e_r - p_r)
        dnn = (e_r - n_r) * (e_r - n_r)
        w_c = d1w_ref[c]
        accp = accp + jnp.dot(dp, w_c, preferred_element_type=f32,
                              precision=lax.Precision.HIGHEST)
        accn = accn + jnp.dot(dnn, w_c, preferred_element_type=f32,
                              precision=lax.Precision.HIGHEST)

    hp = jnp.maximum(accp + d1b_ref[...], 0.0)
    hn = jnp.maximum(accn + d1b_ref[...], 0.0)
    pos_l = jnp.sum(hp * d2w_ref[...], axis=1, keepdims=True) + d2b_ref[0, 0]
    neg_l = jnp.sum(hn * d2w_ref[...], axis=1, keepdims=True) + d2b_ref[0, 0]
    pad = jnp.zeros((B, 126), f32)
    out_ref[...] = jnp.concatenate([pos_l, neg_l, pad], axis=1)


def _prep_call(x, encw, encb, predw_pad, predb_pad, sel2, aidx, w2, src2, idxs):
    f32 = jnp.float32
    i32 = jnp.int32
    vspec = pl.BlockSpec(memory_space=pltpu.MemorySpace.VMEM)
    return pl.pallas_call(
        _prep_kernel,
        out_shape=[
            jax.ShapeDtypeStruct((2, NPAD, HALF), f32),
            jax.ShapeDtypeStruct((N, H), f32),
            jax.ShapeDtypeStruct((N, 8), f32),
            jax.ShapeDtypeStruct((N, 128), f32),
            jax.ShapeDtypeStruct((512, 128), i32),
        ],
        in_specs=[vspec] * 9 + [pl.BlockSpec(memory_space=pltpu.MemorySpace.SMEM)],
        out_specs=[vspec] * 5,
    )(x, encw, encb, predw_pad, predb_pad, sel2, aidx, w2, src2, idxs)


def _hop_call(m, psrc3, pdst3):
    mesh = plsc.VectorSubcoreMesh(core_axis_name="c", subcore_axis_name="s")
    f32 = jnp.float32
    i32 = jnp.int32
    return pl.kernel(
        _hop_kernel,
        out_type=jax.ShapeDtypeStruct((2, N, HALF), f32),
        mesh=mesh,
        scratch_types=[
            pltpu.VMEM_SHARED((4104, HALF), f32),
            pltpu.VMEM_SHARED((N, HALF), f32),
            pltpu.VMEM((CHUNKS_PER_TILE, CHUNK), i32),
            pltpu.VMEM((CHUNKS_PER_TILE, CHUNK), i32),
            pltpu.VMEM((2, CHUNK, HALF), f32),
            pltpu.VMEM((256, HALF), f32),
            pltpu.SemaphoreType.DMA,
            pltpu.SemaphoreType.DMA,
            pltpu.SemaphoreType.DMA,
            pltpu.SemaphoreType.DMA,
        ],
        compiler_params=pltpu.CompilerParams(use_tc_tiling_on_sc=False),
    )(m, psrc3, pdst3)


def _reduce_call(h2, embed, onehot, d1w3, d1b, d2wr, d2b):
    vspec = pl.BlockSpec(memory_space=pltpu.MemorySpace.VMEM)
    return pl.pallas_call(
        _reduce_kernel,
        out_shape=jax.ShapeDtypeStruct((B, 128), jnp.float32),
        in_specs=[vspec] * 7,
        out_specs=vspec,
    )(h2, embed, onehot, d1w3, d1b, d2wr, d2b)


def kernel(x, edge_index, ego, pos, neg, edge_weight, anchor_idx, idx,
           enc_W, enc_b, pred_W, pred_b, dec1_W, dec1_b, dec2_W, dec2_b):
    f32 = jnp.float32
    i32 = jnp.int32
    src2 = edge_index[0].astype(i32).reshape(512, 128)
    dst = edge_index[1].astype(i32)
    w2 = edge_weight.astype(f32).reshape(512, 128)
    sel2 = jnp.concatenate([ego, pos, neg]).astype(i32).reshape(2, HALF)
    aidx = anchor_idx.astype(i32)
    idxs = jnp.asarray(idx, f32).reshape(1, 1)
    predw_pad = jnp.pad(pred_W.astype(f32), ((0, 0), (0, 128 - C)))
    predb_pad = jnp.pad(pred_b.astype(f32), (0, 128 - C)).reshape(1, 128)
    encb = enc_b.astype(f32).reshape(1, H)

    m, embed, onehot, logp_pad, psrc2 = _prep_call(
        x.astype(f32), enc_W.astype(f32), encb, predw_pad, predb_pad, sel2,
        aidx, w2, src2, idxs)

    psrc3 = psrc2.reshape(16, CHUNKS_PER_TILE, CHUNK)
    pdst3 = dst.reshape(16, CHUNKS_PER_TILE, CHUNK)
    h2 = _hop_call(m, psrc3, pdst3)

    d1w3 = dec1_W.astype(f32).reshape(C, H, 32)
    d1b = dec1_b.astype(f32).reshape(1, 32)
    d2wr = dec2_W.astype(f32).reshape(1, 32)
    d2b = dec2_b.astype(f32).reshape(1, 1)
    out2 = _reduce_call(h2, embed, onehot, d1w3, d1b, d2wr, d2b)

    return (out2[:, 0:1], out2[:, 1:2], logp_pad[:, :C])


# speedup vs baseline: 6.6408x; 6.5580x over previous
import jax
import jax.numpy as jnp
from jax import lax
from jax.experimental import pallas as pl
from jax.experimental.pallas import tpu as pltpu
from jax.experimental.pallas import tpu_sc as plsc

N = 4096
E = 65536
D = 128
H = 64
C = 7
B = 64
NA = 32
NCUT = E // 2
SEL = 3 * B
HALF = SEL // 2
NPAD = 4224
NEG_BIG = -1e30

EDGES_PER_TILE = E // 16
CHUNK = 128
CHUNKS_PER_TILE = EDGES_PER_TILE // CHUNK


def _prep_kernel(x_ref, encw_ref, encb_ref, predw_ref, predb_ref, sel_ref,
                 aidx_ref, w_ref, src_ref, idx_ref,
                 m_ref, t_ref, embed_ref, onehot_ref, logp_ref, psrc_ref):
    f32 = jnp.float32
    xw = x_ref[...] + idx_ref[0, 0]
    embed = jnp.maximum(
        jnp.dot(xw, encw_ref[...], preferred_element_type=f32)
        + encb_ref[...], 0.0)
    embed_ref[...] = embed

    logits = (jnp.dot(embed, predw_ref[...], preferred_element_type=f32)
              + predb_ref[...])
    colmask = lax.broadcasted_iota(jnp.int32, (N, 128), 1) < C
    ml = jnp.where(colmask, logits, jnp.float32(NEG_BIG))
    mx = jnp.max(ml, axis=1, keepdims=True)
    z = ml - mx
    s = jnp.sum(jnp.where(colmask, jnp.exp(z), jnp.float32(0.0)), axis=1, keepdims=True)
    logp_ref[...] = z - jnp.log(s)

    iota_n = lax.broadcasted_iota(jnp.int32, (C, NA, N), 2)
    eq = (iota_n == aidx_ref[...][:, :, None]).astype(f32)
    p_mat = jnp.sum(eq, axis=1)
    anchor = jnp.dot(p_mat, embed, preferred_element_type=f32,
                     precision=lax.Precision.HIGHEST) * (1.0 / NA)

    d2_cols = []
    for c in range(C):
        diff = embed - anchor[c:c + 1, :]
        d2_cols.append(jnp.sum(diff * diff, axis=1, keepdims=True))
    d2_cols.append(jnp.full((N, 1), 3e38, f32))
    d2 = jnp.concatenate(d2_cols, axis=1)
    mind = jnp.min(d2, axis=1, keepdims=True)
    iota8 = lax.broadcasted_iota(jnp.int32, (N, 8), 1)
    cand = jnp.where(d2 == mind, iota8, 8)
    cls = jnp.min(cand, axis=1, keepdims=True)
    onehot_ref[...] = (iota8 == cls).astype(f32)

    iota_v = lax.broadcasted_iota(jnp.int32, (2, NPAD, HALF), 1)
    m_vals = (iota_v == sel_ref[...][:, None, :]).astype(f32)
    m_ref[...] = m_vals
    tmax = jnp.max(m_vals, axis=2)
    t_ref[...] = jnp.concatenate(
        [tmax, jnp.zeros((2, 6144 - NPAD), f32)], axis=1)

    w = w_ref[...]

    def bs_body(_, carry):
        lo, hi = carry
        mid = (lo + hi) // 2
        tv = lax.bitcast_convert_type(jnp.full((1, 1), mid, jnp.int32), f32)
        cnt = jnp.sum((w >= tv).astype(f32))
        pred = cnt >= float(NCUT)
        return (jnp.where(pred, mid, lo), jnp.where(pred, hi, mid))

    lo, _ = lax.fori_loop(0, 31, bs_body,
                          (jnp.int32(0), jnp.int32(0x3F800000)))
    thresh = lax.bitcast_convert_type(jnp.full((1, 1), lo, jnp.int32), f32)
    psrc_ref[...] = jnp.where(w >= thresh, src_ref[...], jnp.int32(N))


def _hop_kernel(m_hbm, t_hbm, psrc_hbm, pdst_hbm, out_hbm,
                h1_sh, h2_sh, t1_sh, t1_all_sh, src_v, dst_v, srcc_f, dstc_f,
                t2_v, t1_loc, tl_buf, tm_loc, rows_v, bounce_v, g0, s0):
    i32 = jnp.int32
    f32 = jnp.float32
    cid = lax.axis_index("c")
    sid = lax.axis_index("s")

    @pl.when(sid < 15)
    def _():
        pltpu.sync_copy(m_hbm.at[cid].at[pl.ds(sid * 256, 256)],
                        h1_sh.at[pl.ds(sid * 256, 256)])

    @pl.when(sid == 15)
    def _():
        pltpu.sync_copy(m_hbm.at[cid].at[pl.ds(3840, 264)],
                        h1_sh.at[pl.ds(3840, 264)])

    pltpu.sync_copy(psrc_hbm.at[sid], src_v)
    pltpu.sync_copy(pdst_hbm.at[sid], dst_v)
    pltpu.sync_copy(t_hbm.at[cid], t2_v)

    for r in range(384):
        t1_loc[pl.ds(r * 16, 16)] = jnp.zeros((16,), f32)
    plsc.subcore_barrier()

    def compact(update_t1):
        for r in range(264):
            srcc_f[pl.ds(r * 16, 16)] = jnp.full((16,), N, i32)
            dstc_f[pl.ds(r * 16, 16)] = jnp.zeros((16,), i32)

        def body(k, off):
            row = k // 8
            lane = (k % 8) * 16
            sv = src_v[row, pl.ds(lane, 16)]
            dv = dst_v[row, pl.ds(lane, 16)]
            g = plsc.load_gather(t2_v, [sv])
            msk = g > 0.0
            cum = plsc.cumsum(msk.astype(i32))
            pos = cum - 1 + lax.broadcast(off, (16,))
            plsc.store_scatter(srcc_f, [pos], sv, mask=msk)
            plsc.store_scatter(dstc_f, [pos], dv, mask=msk)
            if update_t1:
                plsc.addupdate_scatter(t1_loc, [dv],
                                       jnp.full((16,), 1.0, f32), mask=msk)
            return off + jnp.max(cum)

        cnt = lax.fori_loop(i32(0), i32(256), body, i32(0))
        return (cnt + 127) // 128

    def run_pass(src_ref, dst_sh, nch):
        def body(j, carry):
            pltpu.async_copy(src_ref.at[srcc_f.at[pl.ds(j * 128, 128)]],
                             rows_v, g0).wait()
            pltpu.async_copy(rows_v, dst_sh.at[dstc_f.at[pl.ds(j * 128, 128)]],
                             s0, add=True).wait()
            return carry

        lax.fori_loop(i32(0), nch, body, i32(0))

    nch1 = compact(update_t1=True)
    run_pass(m_hbm.at[cid], h1_sh, nch1)
    pltpu.sync_copy(t1_loc, t1_all_sh.at[sid])
    plsc.subcore_barrier()

    pltpu.sync_copy(h1_sh.at[pl.ds(sid * 256, 256)], bounce_v)
    pltpu.sync_copy(bounce_v, h2_sh.at[pl.ds(sid * 256, 256)])

    base = sid * 384
    for t in range(16):
        pltpu.sync_copy(t1_all_sh.at[i32(t)].at[pl.ds(base, 384)],
                        tl_buf.at[i32(t)])
    for g in range(24):
        v = t2_v[pl.ds(base + g * 16, 16)]
        for t in range(16):
            v = v + tl_buf[i32(t), pl.ds(g * 16, 16)]
        tm_loc[pl.ds(g * 16, 16)] = v
    pltpu.sync_copy(tm_loc, t1_sh.at[pl.ds(base, 384)])
    plsc.subcore_barrier()

    pltpu.sync_copy(t1_sh, t2_v)
    nch2 = compact(update_t1=False)
    plsc.subcore_barrier()

    run_pass(h1_sh, h2_sh, nch2)
    plsc.subcore_barrier()

    pltpu.sync_copy(h2_sh.at[pl.ds(sid * 256, 256)],
                    out_hbm.at[cid].at[pl.ds(sid * 256, 256)])


def _reduce_kernel(h2_ref, embed_ref, onehot_ref,
                   d1w_ref, d1b_ref, d2w_ref, d2b_ref, out_ref):
    f32 = jnp.float32
    hb0 = (h2_ref[0] > 0.0).astype(f32)
    hb1 = (h2_ref[1] > 0.0).astype(f32)
    embed = embed_ref[...]
    onehot = onehot_ref[...]
    dn = (((0,), (0,)), ((), ()))
    c0 = lax.dot_general(hb0, onehot, dn, preferred_element_type=f32,
                         precision=lax.Precision.HIGHEST)
    c1 = lax.dot_general(hb1, onehot, dn, preferred_element_type=f32,
                         precision=lax.Precision.HIGHEST)
    counts = jnp.concatenate([c0, c1], axis=0)

    accp = jnp.zeros((B, 32), f32)
    accn = jnp.zeros((B, 32), f32)
    for c in range(C):
        wh = embed * onehot[:, c:c + 1]
        s0 = lax.dot_general(hb0, wh, dn, preferred_element_type=f32,
                             precision=lax.Precision.HIGHEST)
        s1 = lax.dot_general(hb1, wh, dn, preferred_element_type=f32,
                             precision=lax.Precision.HIGHEST)
        sums = jnp.concatenate([s0, s1], axis=0)
        means = sums / jnp.maximum(counts[:, c:c + 1], 1.0)
        e_r = means[0:B]
        p_r = means[B:2 * B]
        n_r = means[2 * B:3 * B]
        dp = (e_r - p_r) * (e_r - p_r)
        dnn = (e_r - n_r) * (e_r - n_r)
        w_c = d1w_ref[c]
        accp = accp + jnp.dot(dp, w_c, preferred_element_type=f32,
                              precision=lax.Precision.HIGHEST)
        accn = accn + jnp.dot(dnn, w_c, preferred_element_type=f32,
                              precision=lax.Precision.HIGHEST)

    hp = jnp.maximum(accp + d1b_ref[...], 0.0)
    hn = jnp.maximum(accn + d1b_ref[...], 0.0)
    pos_l = jnp.sum(hp * d2w_ref[...], axis=1, keepdims=True) + d2b_ref[0, 0]
    neg_l = jnp.sum(hn * d2w_ref[...], axis=1, keepdims=True) + d2b_ref[0, 0]
    pad = jnp.zeros((B, 126), f32)
    out_ref[...] = jnp.concatenate([pos_l, neg_l, pad], axis=1)


def _prep_call(x, encw, encb, predw_pad, predb_pad, sel2, aidx, w2, src2, idxs):
    f32 = jnp.float32
    i32 = jnp.int32
    vspec = pl.BlockSpec(memory_space=pltpu.MemorySpace.VMEM)
    return pl.pallas_call(
        _prep_kernel,
        out_shape=[
            jax.ShapeDtypeStruct((2, NPAD, HALF), f32),
            jax.ShapeDtypeStruct((2, 6144), f32),
            jax.ShapeDtypeStruct((N, H), f32),
            jax.ShapeDtypeStruct((N, 8), f32),
            jax.ShapeDtypeStruct((N, 128), f32),
            jax.ShapeDtypeStruct((512, 128), i32),
        ],
        in_specs=[vspec] * 9 + [pl.BlockSpec(memory_space=pltpu.MemorySpace.SMEM)],
        out_specs=[vspec] * 6,
    )(x, encw, encb, predw_pad, predb_pad, sel2, aidx, w2, src2, idxs)


def _hop_call(m, t, psrc3, pdst3):
    mesh = plsc.VectorSubcoreMesh(core_axis_name="c", subcore_axis_name="s")
    f32 = jnp.float32
    i32 = jnp.int32
    return pl.kernel(
        _hop_kernel,
        out_type=jax.ShapeDtypeStruct((2, N, HALF), f32),
        mesh=mesh,
        scratch_types=[
            pltpu.VMEM_SHARED((4104, HALF), f32),
            pltpu.VMEM_SHARED((N, HALF), f32),
            pltpu.VMEM_SHARED((6144,), f32),
            pltpu.VMEM_SHARED((16, 6144), f32),
            pltpu.VMEM((CHUNKS_PER_TILE, CHUNK), i32),
            pltpu.VMEM((CHUNKS_PER_TILE, CHUNK), i32),
            pltpu.VMEM((4224,), i32),
            pltpu.VMEM((4224,), i32),
            pltpu.VMEM((6144,), f32),
            pltpu.VMEM((6144,), f32),
            pltpu.VMEM((16, 384), f32),
            pltpu.VMEM((384,), f32),
            pltpu.VMEM((CHUNK, HALF), f32),
            pltpu.VMEM((256, HALF), f32),
            pltpu.SemaphoreType.DMA,
            pltpu.SemaphoreType.DMA,
        ],
        compiler_params=pltpu.CompilerParams(use_tc_tiling_on_sc=False,
                                             needs_layout_passes=False),
    )(m, t, psrc3, pdst3)


def _reduce_call(h2, embed, onehot, d1w3, d1b, d2wr, d2b):
    vspec = pl.BlockSpec(memory_space=pltpu.MemorySpace.VMEM)
    return pl.pallas_call(
        _reduce_kernel,
        out_shape=jax.ShapeDtypeStruct((B, 128), jnp.float32),
        in_specs=[vspec] * 7,
        out_specs=vspec,
    )(h2, embed, onehot, d1w3, d1b, d2wr, d2b)


def kernel(x, edge_index, ego, pos, neg, edge_weight, anchor_idx, idx,
           enc_W, enc_b, pred_W, pred_b, dec1_W, dec1_b, dec2_W, dec2_b):
    f32 = jnp.float32
    i32 = jnp.int32
    src2 = edge_index[0].astype(i32).reshape(512, 128)
    dst = edge_index[1].astype(i32)
    w2 = edge_weight.astype(f32).reshape(512, 128)
    sel2 = jnp.concatenate([ego, pos, neg]).astype(i32).reshape(2, HALF)
    aidx = anchor_idx.astype(i32)
    idxs = jnp.asarray(idx, f32).reshape(1, 1)
    predw_pad = jnp.pad(pred_W.astype(f32), ((0, 0), (0, 128 - C)))
    predb_pad = jnp.pad(pred_b.astype(f32), (0, 128 - C)).reshape(1, 128)
    encb = enc_b.astype(f32).reshape(1, H)

    m, t, embed, onehot, logp_pad, psrc2 = _prep_call(
        x.astype(f32), enc_W.astype(f32), encb, predw_pad, predb_pad, sel2,
        aidx, w2, src2, idxs)

    psrc3 = psrc2.reshape(16, CHUNKS_PER_TILE, CHUNK)
    pdst3 = dst.reshape(16, CHUNKS_PER_TILE, CHUNK)
    h2 = _hop_call(m, t, psrc3, pdst3)

    d1w3 = dec1_W.astype(f32).reshape(C, H, 32)
    d1b = dec1_b.astype(f32).reshape(1, 32)
    d2wr = dec2_W.astype(f32).reshape(1, 32)
    d2b = dec2_b.astype(f32).reshape(1, 1)
    out2 = _reduce_call(h2, embed, onehot, d1w3, d1b, d2wr, d2b)

    return (out2[:, 0:1], out2[:, 1:2], logp_pad[:, :C])


# speedup vs baseline: 7.4983x; 1.1291x over previous
import jax
import jax.numpy as jnp
from jax import lax
from jax.experimental import pallas as pl
from jax.experimental.pallas import tpu as pltpu
from jax.experimental.pallas import tpu_sc as plsc

N = 4096
E = 65536
D = 128
H = 64
C = 7
B = 64
NA = 32
NCUT = E // 2
SEL = 3 * B
HALF = SEL // 2
NPAD = 4224
NEG_BIG = -1e30

EDGES_PER_TILE = E // 16
CHUNK = 128
CHUNKS_PER_TILE = EDGES_PER_TILE // CHUNK


def _prep_sc_kernel(sel_ref, w_ref, src_ref,
                    m_ref, t_ref, psrc_ref):
    f32 = jnp.float32
    iota_v = lax.broadcasted_iota(jnp.int32, (2, NPAD, HALF), 1)
    m_vals = (iota_v == sel_ref[...][:, None, :]).astype(f32)
    m_ref[...] = m_vals
    tmax = jnp.max(m_vals, axis=2)
    t_ref[...] = jnp.concatenate(
        [tmax, jnp.zeros((2, 6144 - NPAD), f32)], axis=1)

    w = w_ref[...]

    def bs_body(_, carry):
        lo, hi = carry
        mid = (lo + hi) // 2
        tv = lax.bitcast_convert_type(jnp.full((1, 1), mid, jnp.int32), f32)
        cnt = jnp.sum((w >= tv).astype(f32))
        pred = cnt >= float(NCUT)
        return (jnp.where(pred, mid, lo), jnp.where(pred, hi, mid))

    lo, _ = lax.fori_loop(0, 31, bs_body,
                          (jnp.int32(0), jnp.int32(0x3F800000)))
    thresh = lax.bitcast_convert_type(jnp.full((1, 1), lo, jnp.int32), f32)
    psrc_ref[...] = jnp.where(w >= thresh, src_ref[...], jnp.int32(N))


def _prep_tc_kernel(x_ref, encw_ref, encb_ref, predw_ref, predb_ref,
                    aidx_ref, idx_ref,
                    embed_ref, onehot_ref, logp_ref):
    f32 = jnp.float32
    xw = x_ref[...] + idx_ref[0, 0]
    embed = jnp.maximum(
        jnp.dot(xw, encw_ref[...], preferred_element_type=f32)
        + encb_ref[...], 0.0)
    embed_ref[...] = embed

    logits = (jnp.dot(embed, predw_ref[...], preferred_element_type=f32)
              + predb_ref[...])
    colmask = lax.broadcasted_iota(jnp.int32, (N, 128), 1) < C
    ml = jnp.where(colmask, logits, jnp.float32(NEG_BIG))
    mx = jnp.max(ml, axis=1, keepdims=True)
    z = ml - mx
    s = jnp.sum(jnp.where(colmask, jnp.exp(z), jnp.float32(0.0)),
                axis=1, keepdims=True)
    logp_ref[...] = z - jnp.log(s)

    iota_n = lax.broadcasted_iota(jnp.int32, (C, NA, N), 2)
    eq = (iota_n == aidx_ref[...][:, :, None]).astype(f32)
    p_mat = jnp.sum(eq, axis=1)
    anchor = jnp.dot(p_mat, embed, preferred_element_type=f32,
                     precision=lax.Precision.HIGHEST) * (1.0 / NA)

    d2_cols = []
    for c in range(C):
        diff = embed - anchor[c:c + 1, :]
        d2_cols.append(jnp.sum(diff * diff, axis=1, keepdims=True))
    d2_cols.append(jnp.full((N, 1), 3e38, f32))
    d2 = jnp.concatenate(d2_cols, axis=1)
    mind = jnp.min(d2, axis=1, keepdims=True)
    iota8 = lax.broadcasted_iota(jnp.int32, (N, 8), 1)
    cand = jnp.where(d2 == mind, iota8, 8)
    cls = jnp.min(cand, axis=1, keepdims=True)
    onehot_ref[...] = (iota8 == cls).astype(f32)


def _hop_kernel(m_hbm, t_hbm, psrc_hbm, pdst_hbm, out_hbm,
                h1_sh, h2_sh, t1_sh, t1_all_sh, src_v, dst_v, srcc_f, dstc_f,
                t2_v, t1_loc, tl_buf, tm_loc, rows_v, bounce_v, g0, s0):
    i32 = jnp.int32
    f32 = jnp.float32
    cid = lax.axis_index("c")
    sid = lax.axis_index("s")

    @pl.when(sid < 15)
    def _():
        pltpu.sync_copy(m_hbm.at[cid].at[pl.ds(sid * 256, 256)],
                        h1_sh.at[pl.ds(sid * 256, 256)])

    @pl.when(sid == 15)
    def _():
        pltpu.sync_copy(m_hbm.at[cid].at[pl.ds(3840, 264)],
                        h1_sh.at[pl.ds(3840, 264)])

    pltpu.sync_copy(psrc_hbm.at[sid], src_v)
    pltpu.sync_copy(pdst_hbm.at[sid], dst_v)
    pltpu.sync_copy(t_hbm.at[cid], t2_v)

    for r in range(384):
        t1_loc[pl.ds(r * 16, 16)] = jnp.zeros((16,), f32)
    plsc.subcore_barrier()

    def compact(update_t1):
        for r in range(264):
            srcc_f[pl.ds(r * 16, 16)] = jnp.full((16,), N, i32)
            dstc_f[pl.ds(r * 16, 16)] = jnp.zeros((16,), i32)

        def body(k, off):
            row = k // 8
            lane = (k % 8) * 16
            sv = src_v[row, pl.ds(lane, 16)]
            dv = dst_v[row, pl.ds(lane, 16)]
            g = plsc.load_gather(t2_v, [sv])
            msk = g > 0.0
            cum = plsc.cumsum(msk.astype(i32))
            pos = cum - 1 + lax.broadcast(off, (16,))
            plsc.store_scatter(srcc_f, [pos], sv, mask=msk)
            plsc.store_scatter(dstc_f, [pos], dv, mask=msk)
            if update_t1:
                plsc.addupdate_scatter(t1_loc, [dv],
                                       jnp.full((16,), 1.0, f32), mask=msk)
            return off + jnp.max(cum)

        cnt = lax.fori_loop(i32(0), i32(256), body, i32(0))
        return (cnt + 127) // 128

    def run_pass(src_ref, dst_sh, nch):
        def body(j, carry):
            pltpu.async_copy(src_ref.at[srcc_f.at[pl.ds(j * 128, 128)]],
                             rows_v, g0).wait()
            pltpu.async_copy(rows_v, dst_sh.at[dstc_f.at[pl.ds(j * 128, 128)]],
                             s0, add=True).wait()
            return carry

        lax.fori_loop(i32(0), nch, body, i32(0))

    nch1 = compact(update_t1=True)
    run_pass(m_hbm.at[cid], h1_sh, nch1)
    pltpu.sync_copy(t1_loc, t1_all_sh.at[sid])
    plsc.subcore_barrier()

    pltpu.sync_copy(h1_sh.at[pl.ds(sid * 256, 256)], bounce_v)
    pltpu.sync_copy(bounce_v, h2_sh.at[pl.ds(sid * 256, 256)])

    base = sid * 384
    for t in range(16):
        pltpu.sync_copy(t1_all_sh.at[i32(t)].at[pl.ds(base, 384)],
                        tl_buf.at[i32(t)])
    for g in range(24):
        v = t2_v[pl.ds(base + g * 16, 16)]
        for t in range(16):
            v = v + tl_buf[i32(t), pl.ds(g * 16, 16)]
        tm_loc[pl.ds(g * 16, 16)] = v
    pltpu.sync_copy(tm_loc, t1_sh.at[pl.ds(base, 384)])
    plsc.subcore_barrier()

    pltpu.sync_copy(t1_sh, t2_v)
    nch2 = compact(update_t1=False)
    plsc.subcore_barrier()

    run_pass(h1_sh, h2_sh, nch2)
    plsc.subcore_barrier()

    pltpu.sync_copy(h2_sh.at[pl.ds(sid * 256, 256)],
                    out_hbm.at[cid].at[pl.ds(sid * 256, 256)])


def _reduce_kernel(h2_ref, embed_ref, onehot_ref,
                   d1w_ref, d1b_ref, d2w_ref, d2b_ref, out_ref):
    f32 = jnp.float32
    hb0 = (h2_ref[0] > 0.0).astype(f32)
    hb1 = (h2_ref[1] > 0.0).astype(f32)
    embed = embed_ref[...]
    onehot = onehot_ref[...]
    dn = (((0,), (0,)), ((), ()))
    c0 = lax.dot_general(hb0, onehot, dn, preferred_element_type=f32,
                         precision=lax.Precision.HIGHEST)
    c1 = lax.dot_general(hb1, onehot, dn, preferred_element_type=f32,
                         precision=lax.Precision.HIGHEST)
    counts = jnp.concatenate([c0, c1], axis=0)

    accp = jnp.zeros((B, 32), f32)
    accn = jnp.zeros((B, 32), f32)
    for c in range(C):
        wh = embed * onehot[:, c:c + 1]
        s0 = lax.dot_general(hb0, wh, dn, preferred_element_type=f32,
                             precision=lax.Precision.HIGHEST)
        s1 = lax.dot_general(hb1, wh, dn, preferred_element_type=f32,
                             precision=lax.Precision.HIGHEST)
        sums = jnp.concatenate([s0, s1], axis=0)
        means = sums / jnp.maximum(counts[:, c:c + 1], 1.0)
        e_r = means[0:B]
        p_r = means[B:2 * B]
        n_r = means[2 * B:3 * B]
        dp = (e_r - p_r) * (e_r - p_r)
        dnn = (e_r - n_r) * (e_r - n_r)
        w_c = d1w_ref[c]
        accp = accp + jnp.dot(dp, w_c, preferred_element_type=f32,
                              precision=lax.Precision.HIGHEST)
        accn = accn + jnp.dot(dnn, w_c, preferred_element_type=f32,
                              precision=lax.Precision.HIGHEST)

    hp = jnp.maximum(accp + d1b_ref[...], 0.0)
    hn = jnp.maximum(accn + d1b_ref[...], 0.0)
    pos_l = jnp.sum(hp * d2w_ref[...], axis=1, keepdims=True) + d2b_ref[0, 0]
    neg_l = jnp.sum(hn * d2w_ref[...], axis=1, keepdims=True) + d2b_ref[0, 0]
    pad = jnp.zeros((B, 126), f32)
    out_ref[...] = jnp.concatenate([pos_l, neg_l, pad], axis=1)


def _prep_sc_call(sel2, w2, src2):
    f32 = jnp.float32
    i32 = jnp.int32
    vspec = pl.BlockSpec(memory_space=pltpu.MemorySpace.VMEM)
    return pl.pallas_call(
        _prep_sc_kernel,
        out_shape=[
            jax.ShapeDtypeStruct((2, NPAD, HALF), f32),
            jax.ShapeDtypeStruct((2, 6144), f32),
            jax.ShapeDtypeStruct((512, 128), i32),
        ],
        in_specs=[vspec] * 3,
        out_specs=[vspec] * 3,
    )(sel2, w2, src2)


def _prep_tc_call(x, encw, encb, predw_pad, predb_pad, aidx, idxs):
    f32 = jnp.float32
    vspec = pl.BlockSpec(memory_space=pltpu.MemorySpace.VMEM)
    return pl.pallas_call(
        _prep_tc_kernel,
        out_shape=[
            jax.ShapeDtypeStruct((N, H), f32),
            jax.ShapeDtypeStruct((N, 8), f32),
            jax.ShapeDtypeStruct((N, 128), f32),
        ],
        in_specs=[vspec] * 6 + [pl.BlockSpec(memory_space=pltpu.MemorySpace.SMEM)],
        out_specs=[vspec] * 3,
    )(x, encw, encb, predw_pad, predb_pad, aidx, idxs)


def _hop_call(m, t, psrc3, pdst3):
    mesh = plsc.VectorSubcoreMesh(core_axis_name="c", subcore_axis_name="s")
    f32 = jnp.float32
    i32 = jnp.int32
    return pl.kernel(
        _hop_kernel,
        out_type=jax.ShapeDtypeStruct((2, N, HALF), f32),
        mesh=mesh,
        scratch_types=[
            pltpu.VMEM_SHARED((4104, HALF), f32),
            pltpu.VMEM_SHARED((N, HALF), f32),
            pltpu.VMEM_SHARED((6144,), f32),
            pltpu.VMEM_SHARED((16, 6144), f32),
            pltpu.VMEM((CHUNKS_PER_TILE, CHUNK), i32),
            pltpu.VMEM((CHUNKS_PER_TILE, CHUNK), i32),
            pltpu.VMEM((4224,), i32),
            pltpu.VMEM((4224,), i32),
            pltpu.VMEM((6144,), f32),
            pltpu.VMEM((6144,), f32),
            pltpu.VMEM((16, 384), f32),
            pltpu.VMEM((384,), f32),
            pltpu.VMEM((CHUNK, HALF), f32),
            pltpu.VMEM((256, HALF), f32),
            pltpu.SemaphoreType.DMA,
            pltpu.SemaphoreType.DMA,
        ],
        compiler_params=pltpu.CompilerParams(use_tc_tiling_on_sc=False,
                                             needs_layout_passes=False),
    )(m, t, psrc3, pdst3)


def _reduce_call(h2, embed, onehot, d1w3, d1b, d2wr, d2b):
    vspec = pl.BlockSpec(memory_space=pltpu.MemorySpace.VMEM)
    return pl.pallas_call(
        _reduce_kernel,
        out_shape=jax.ShapeDtypeStruct((B, 128), jnp.float32),
        in_specs=[vspec] * 7,
        out_specs=vspec,
    )(h2, embed, onehot, d1w3, d1b, d2wr, d2b)


def kernel(x, edge_index, ego, pos, neg, edge_weight, anchor_idx, idx,
           enc_W, enc_b, pred_W, pred_b, dec1_W, dec1_b, dec2_W, dec2_b):
    f32 = jnp.float32
    i32 = jnp.int32
    src2 = edge_index[0].astype(i32).reshape(512, 128)
    dst = edge_index[1].astype(i32)
    w2 = edge_weight.astype(f32).reshape(512, 128)
    sel2 = jnp.concatenate([ego, pos, neg]).astype(i32).reshape(2, HALF)
    aidx = anchor_idx.astype(i32)
    idxs = jnp.asarray(idx, f32).reshape(1, 1)
    predw_pad = jnp.pad(pred_W.astype(f32), ((0, 0), (0, 128 - C)))
    predb_pad = jnp.pad(pred_b.astype(f32), (0, 128 - C)).reshape(1, 128)
    encb = enc_b.astype(f32).reshape(1, H)

    m, t, psrc2 = _prep_sc_call(sel2, w2, src2)
    embed, onehot, logp_pad = _prep_tc_call(
        x.astype(f32), enc_W.astype(f32), encb, predw_pad, predb_pad, aidx,
        idxs)

    psrc3 = psrc2.reshape(16, CHUNKS_PER_TILE, CHUNK)
    pdst3 = dst.reshape(16, CHUNKS_PER_TILE, CHUNK)
    h2 = _hop_call(m, t, psrc3, pdst3)

    d1w3 = dec1_W.astype(f32).reshape(C, H, 32)
    d1b = dec1_b.astype(f32).reshape(1, 32)
    d2wr = dec2_W.astype(f32).reshape(1, 32)
    d2b = dec2_b.astype(f32).reshape(1, 1)
    out2 = _reduce_call(h2, embed, onehot, d1w3, d1b, d2wr, d2b)

    return (out2[:, 0:1], out2[:, 1:2], logp_pad[:, :C])
